# baseline (speedup 1.0000x reference)
import functools

import jax
import jax.numpy as jnp
from jax import lax
from jax.experimental import pallas as pl
from jax.experimental.pallas import tpu as pltpu

_TOPK = 16


def _body(fv_ref, ctx_ref, wq_ref, wk_ref, wv_ref, out_ref, idx_ref, vals_ref):
    fv = fv_ref[0]
    ctx = ctx_ref[0]
    n, d = fv.shape
    m = ctx.shape[0]

    q = jnp.dot(fv, wq_ref[...], preferred_element_type=jnp.float32)
    kk = jnp.dot(ctx, wk_ref[...], preferred_element_type=jnp.float32)
    vv = jnp.dot(ctx, wv_ref[...], preferred_element_type=jnp.float32)
    kk = kk - jnp.max(kk, axis=0, keepdims=True)
    ek = jnp.exp(kk)
    kk = ek / jnp.sum(ek, axis=0, keepdims=True)
    lam = lax.dot_general(kk, vv, (((0,), (0,)), ((), ())),
                          preferred_element_type=jnp.float32)
    w = jnp.dot(q, lam, preferred_element_type=jnp.float32)

    iota_f = lax.broadcasted_iota(jnp.int32, (n, m), 1).astype(jnp.float32)
    for t in range(_TOPK):
        mx = jnp.max(w, axis=1, keepdims=True)
        am = jnp.min(jnp.where(w == mx, iota_f, float(m)), axis=1,
                     keepdims=True)
        idx_ref[:, t] = am[:, 0]
        vals_ref[:, t] = mx[:, 0]
        w = jnp.where(iota_f == am, -jnp.inf, w)

    vals = vals_ref[...]
    sm = jnp.exp(vals - jnp.max(vals, axis=1, keepdims=True))
    sm = sm / jnp.sum(sm, axis=1, keepdims=True)

    iota3 = lax.broadcasted_iota(jnp.int32, (n, _TOPK, m), 2).astype(jnp.float32)
    sc = jnp.where(iota3 == idx_ref[...][:, :, None],
                   sm[:, :, None], 0.0)
    g = jnp.dot(sc.reshape(n * _TOPK, m), ctx,
                preferred_element_type=jnp.float32)
    out = g.reshape(n, _TOPK, d) * fv[:, None, :]
    out_ref[0] = out.reshape(n * _TOPK, d)


_NSPLIT = 4


@jax.jit
def kernel(featureVec, contextVec, Wq, Wk, Wv):
    b, n, d = featureVec.shape
    m = contextVec.shape[1]
    nc = n // _NSPLIT
    return pl.pallas_call(
        _body,
        grid=(b, _NSPLIT),
        in_specs=[
            pl.BlockSpec((1, nc, d), lambda i, j: (i, j, 0)),
            pl.BlockSpec((1, m, d), lambda i, j: (i, 0, 0)),
            pl.BlockSpec((d, d), lambda i, j: (0, 0)),
            pl.BlockSpec((d, d), lambda i, j: (0, 0)),
            pl.BlockSpec((d, m), lambda i, j: (0, 0)),
        ],
        out_specs=pl.BlockSpec((1, nc * _TOPK, d), lambda i, j: (i, j, 0)),
        out_shape=jax.ShapeDtypeStruct((b, n * _TOPK, d), jnp.float32),
        scratch_shapes=[
            pltpu.VMEM((nc, _TOPK), jnp.float32),
            pltpu.VMEM((nc, _TOPK), jnp.float32),
        ],
    )(featureVec, contextVec, Wq, Wk, Wv)


# speedup vs baseline: 1.7004x; 1.7004x over previous
import functools

import jax
import jax.numpy as jnp
from jax import lax
from jax.experimental import pallas as pl
from jax.experimental.pallas import tpu as pltpu

_TOPK = 16


def _body(fv_ref, ctx_ref, wq_ref, wk_ref, wv_ref, out_ref, idx_ref, vals_ref):
    fv = fv_ref[0]
    ctx = ctx_ref[0]
    n, d = fv.shape
    m = ctx.shape[0]

    q = jnp.dot(fv, wq_ref[...], preferred_element_type=jnp.float32)
    kk = jnp.dot(ctx, wk_ref[...], preferred_element_type=jnp.float32)
    vv = jnp.dot(ctx, wv_ref[...], preferred_element_type=jnp.float32)
    kk = kk - jnp.max(kk, axis=0, keepdims=True)
    ek = jnp.exp(kk)
    kk = ek / jnp.sum(ek, axis=0, keepdims=True)
    lam = lax.dot_general(kk, vv, (((0,), (0,)), ((), ())),
                          preferred_element_type=jnp.float32)
    w = jnp.dot(q, lam, preferred_element_type=jnp.float32)

    iota_f = lax.broadcasted_iota(jnp.int32, (n, m), 1).astype(jnp.float32)
    for t in range(_TOPK):
        mx = jnp.max(w, axis=1, keepdims=True)
        am = jnp.min(jnp.where(w == mx, iota_f, float(m)), axis=1,
                     keepdims=True)
        idx_ref[:, t] = am[:, 0]
        vals_ref[:, t] = mx[:, 0]
        w = jnp.where(iota_f == am, -jnp.inf, w)

    vals = vals_ref[...]
    sm = jnp.exp(vals - jnp.max(vals, axis=1, keepdims=True))
    sm = sm / jnp.sum(sm, axis=1, keepdims=True)

    iota3 = lax.broadcasted_iota(jnp.int32, (n, _TOPK, m), 2).astype(jnp.float32)
    sc = jnp.where(iota3 == idx_ref[...][:, :, None],
                   sm[:, :, None], 0.0)
    g = jnp.dot(sc.reshape(n * _TOPK, m), ctx,
                preferred_element_type=jnp.float32)
    out = g.reshape(n, _TOPK, d) * fv[:, None, :]
    out_ref[0] = out.reshape(n * _TOPK, d)


_NSPLIT = 2


@jax.jit
def kernel(featureVec, contextVec, Wq, Wk, Wv):
    b, n, d = featureVec.shape
    m = contextVec.shape[1]
    nc = n // _NSPLIT
    return pl.pallas_call(
        _body,
        grid=(b, _NSPLIT),
        in_specs=[
            pl.BlockSpec((1, nc, d), lambda i, j: (i, j, 0)),
            pl.BlockSpec((1, m, d), lambda i, j: (i, 0, 0)),
            pl.BlockSpec((d, d), lambda i, j: (0, 0)),
            pl.BlockSpec((d, d), lambda i, j: (0, 0)),
            pl.BlockSpec((d, m), lambda i, j: (0, 0)),
        ],
        out_specs=pl.BlockSpec((1, nc * _TOPK, d), lambda i, j: (i, j, 0)),
        out_shape=jax.ShapeDtypeStruct((b, n * _TOPK, d), jnp.float32),
        scratch_shapes=[
            pltpu.VMEM((nc, _TOPK), jnp.float32),
            pltpu.VMEM((nc, _TOPK), jnp.float32),
        ],
    )(featureVec, contextVec, Wq, Wk, Wv)


# speedup vs baseline: 2.0611x; 1.2121x over previous
import functools

import jax
import jax.numpy as jnp
from jax import lax
from jax.experimental import pallas as pl
from jax.experimental.pallas import tpu as pltpu

_TOPK = 16


def _body(fv_ref, ctx_ref, wq_ref, wk_ref, wv_ref, out_ref, idx_ref, vals_ref):
    fv = fv_ref[0]
    ctx = ctx_ref[0]
    n, d = fv.shape
    m = ctx.shape[0]

    q = jnp.dot(fv, wq_ref[...], preferred_element_type=jnp.float32)
    kk = jnp.dot(ctx, wk_ref[...], preferred_element_type=jnp.float32)
    vv = jnp.dot(ctx, wv_ref[...], preferred_element_type=jnp.float32)
    kk = kk - jnp.max(kk, axis=0, keepdims=True)
    ek = jnp.exp(kk)
    kk = ek / jnp.sum(ek, axis=0, keepdims=True)
    lam = lax.dot_general(kk, vv, (((0,), (0,)), ((), ())),
                          preferred_element_type=jnp.float32)
    w = jnp.dot(q, lam, preferred_element_type=jnp.float32)

    iota_f = lax.broadcasted_iota(jnp.int32, (n, m), 1).astype(jnp.float32)
    for t in range(_TOPK):
        mx = jnp.max(w, axis=1, keepdims=True)
        am = jnp.min(jnp.where(w == mx, iota_f, float(m)), axis=1,
                     keepdims=True)
        idx_ref[:, t] = am[:, 0]
        vals_ref[:, t] = mx[:, 0]
        w = jnp.where(iota_f == am, -jnp.inf, w)

    vals = vals_ref[...]
    sm = jnp.exp(vals - jnp.max(vals, axis=1, keepdims=True))
    sm = sm / jnp.sum(sm, axis=1, keepdims=True)

    iota3 = lax.broadcasted_iota(jnp.int32, (n, _TOPK, m), 2).astype(jnp.bfloat16)
    idx_b = idx_ref[...].astype(jnp.bfloat16)
    sc = jnp.where(iota3 == idx_b[:, :, None],
                   sm.astype(jnp.bfloat16)[:, :, None],
                   jnp.bfloat16(0.0))
    g = jnp.dot(sc.reshape(n * _TOPK, m), ctx.astype(jnp.bfloat16),
                preferred_element_type=jnp.float32)
    out = g.reshape(n, _TOPK, d) * fv[:, None, :]
    out_ref[0] = out.reshape(n * _TOPK, d)


@jax.jit
def kernel(featureVec, contextVec, Wq, Wk, Wv):
    b, n, d = featureVec.shape
    m = contextVec.shape[1]
    return pl.pallas_call(
        _body,
        grid=(b,),
        in_specs=[
            pl.BlockSpec((1, n, d), lambda i: (i, 0, 0)),
            pl.BlockSpec((1, m, d), lambda i: (i, 0, 0)),
            pl.BlockSpec((d, d), lambda i: (0, 0)),
            pl.BlockSpec((d, d), lambda i: (0, 0)),
            pl.BlockSpec((d, m), lambda i: (0, 0)),
        ],
        out_specs=pl.BlockSpec((1, n * _TOPK, d), lambda i: (i, 0, 0)),
        out_shape=jax.ShapeDtypeStruct((b, n * _TOPK, d), jnp.float32),
        scratch_shapes=[
            pltpu.VMEM((n, _TOPK), jnp.float32),
            pltpu.VMEM((n, _TOPK), jnp.float32),
        ],
    )(featureVec, contextVec, Wq, Wk, Wv)
